# Initial kernel scaffold; baseline (speedup 1.0000x reference)
import jax
import jax.numpy as jnp
from jax.experimental import pallas as pl


def kernel(input, char_table, char_emb, tok_emb):
    raise NotImplementedError("write your pallas kernel here")


# speedup vs baseline: 15.7625x; 15.7625x over previous
import functools
import math

import jax
import jax.numpy as jnp
from jax import lax
from jax.experimental import pallas as pl
from jax.experimental.pallas import tpu as pltpu
from jax.experimental.pallas import tpu_sc as plsc

D = 128
M = 16
C = 256
ROPE_BASE = 10000.0


def _rot_table_body(emb_ref, out_ref):
    m = pl.program_id(0).astype(jnp.float32)
    e = emb_ref[...]
    col = lax.broadcasted_iota(jnp.int32, (C, D), 1)
    two_k = (col - (col % 2)).astype(jnp.float32)
    freq = jnp.exp(two_k * (-math.log(ROPE_BASE) / D))
    ang = m * freq
    cosv = jnp.cos(ang)
    sinv = jnp.sin(ang)
    left = jnp.concatenate([e[:, 1:], e[:, :1]], axis=1)
    right = jnp.concatenate([e[:, -1:], e[:, :-1]], axis=1)
    odd = (col % 2) == 1
    swap = jnp.where(odd, right, left)
    sign = jnp.where(odd, 1.0, -1.0)
    out_ref[...] = e * cosv + swap * sinv * sign


def _rot_table(char_emb):
    return pl.pallas_call(
        _rot_table_body,
        grid=(M,),
        in_specs=[pl.BlockSpec((C, D), lambda m: (0, 0))],
        out_specs=pl.BlockSpec((C, D), lambda m: (m, 0)),
        out_shape=jax.ShapeDtypeStruct((M * C, D), jnp.float32),
    )(char_emb)


def _sc_lookup(ids, char_table, rot, tok_emb):
    n = ids.shape[0]
    info = plsc.get_sparse_core_info()
    nw = info.num_cores * info.num_subcores
    per_w = n // nw
    T = 32
    nchunk = per_w // T

    mesh = plsc.VectorSubcoreMesh(core_axis_name="c", subcore_axis_name="s")

    @functools.partial(
        pl.kernel,
        out_type=jax.ShapeDtypeStruct((n, D), jnp.float32),
        mesh=mesh,
        scratch_types=[
            pltpu.VMEM((per_w,), jnp.int32),
            pltpu.VMEM((T * M,), jnp.int32),
            pltpu.VMEM((T * M,), jnp.int32),
            pltpu.VMEM((T, D), jnp.float32),
            pltpu.VMEM((T * M, D), jnp.float32),
            pltpu.SemaphoreType.DMA,
            pltpu.SemaphoreType.DMA,
        ],
    )
    def k(ids_hbm, chart_hbm, rot_hbm, tok_hbm, out_hbm,
          ids_v, cidx_v, rc_v, acc_v, rot_buf, sem_a, sem_b):
        wid = lax.axis_index("s") * info.num_cores + lax.axis_index("c")
        base = wid * per_w
        pltpu.sync_copy(ids_hbm.at[pl.ds(base, per_w)], ids_v)
        lane = lax.iota(jnp.int32, 16)
        offs = lane * C

        def chunk_body(c, carry):
            off = c * T
            idx = ids_v.at[pl.ds(off, T)]
            d_tok = pltpu.async_copy(tok_hbm.at[idx], acc_v, sem_a)

            def cidx_body(g, carry2):
                v = ids_v[pl.ds(off + g * 16, 16)] * M
                for t in range(16):
                    cidx_v[pl.ds((g * 16 + t) * M, M)] = v[t] + lane
                return carry2

            lax.fori_loop(0, T // 16, cidx_body, 0)
            ch_descs = [
                pltpu.async_copy(
                    chart_hbm.at[cidx_v.at[pl.ds(i * 128, 128)]],
                    rc_v.at[pl.ds(i * 128, 128)],
                    sem_b,
                )
                for i in range(T * M // 128)
            ]
            for dd in ch_descs:
                dd.wait()

            def rc_body(v, carry2):
                sl = pl.ds(v * 16, 16)
                rc_v[sl] = rc_v[sl] + offs
                return carry2

            lax.fori_loop(0, T * M // 16, rc_body, 0)
            descs = [
                pltpu.async_copy(
                    rot_hbm.at[rc_v.at[pl.ds(i * 128, 128)]],
                    rot_buf.at[pl.ds(i * 128, 128)],
                    sem_b,
                )
                for i in range(T * M // 128)
            ]
            d_tok.wait()
            for dd in descs:
                dd.wait()

            def tok_body(j, carry2):
                rbase = j * M
                for kk in range(D // 16):
                    sl = pl.ds(kk * 16, 16)
                    a = acc_v[j, sl]
                    for m in range(M):
                        a = a + rot_buf[rbase + m, sl]
                    acc_v[j, sl] = a
                return carry2

            lax.fori_loop(0, T, tok_body, 0)
            pltpu.sync_copy(acc_v, out_hbm.at[pl.ds(base + off, T)])
            return carry

        lax.fori_loop(0, nchunk, chunk_body, 0)

    return k(ids, char_table, rot, tok_emb)


def kernel(input, char_table, char_emb, tok_emb):
    b, s = input.shape
    rot = _rot_table(char_emb)
    ids = input.reshape(-1)
    out = _sc_lookup(ids, char_table.reshape(-1), rot, tok_emb)
    return out.reshape(b, s, D)


# speedup vs baseline: 21.8899x; 1.3887x over previous
import functools
import math

import jax
import jax.numpy as jnp
from jax import lax
from jax.experimental import pallas as pl
from jax.experimental.pallas import tpu as pltpu
from jax.experimental.pallas import tpu_sc as plsc

D = 128
M = 16
C = 256
ROPE_BASE = 10000.0


def _rot_table_body(emb_ref, out_ref):
    m = pl.program_id(0).astype(jnp.float32)
    e = emb_ref[...]
    col = lax.broadcasted_iota(jnp.int32, (C, D), 1)
    two_k = (col - (col % 2)).astype(jnp.float32)
    freq = jnp.exp(two_k * (-math.log(ROPE_BASE) / D))
    ang = m * freq
    cosv = jnp.cos(ang)
    sinv = jnp.sin(ang)
    left = jnp.concatenate([e[:, 1:], e[:, :1]], axis=1)
    right = jnp.concatenate([e[:, -1:], e[:, :-1]], axis=1)
    odd = (col % 2) == 1
    swap = jnp.where(odd, right, left)
    sign = jnp.where(odd, 1.0, -1.0)
    out_ref[...] = e * cosv + swap * sinv * sign


def _rot_table(char_emb):
    return pl.pallas_call(
        _rot_table_body,
        grid=(M,),
        in_specs=[pl.BlockSpec((C, D), lambda m: (0, 0))],
        out_specs=pl.BlockSpec((C, D), lambda m: (m, 0)),
        out_shape=jax.ShapeDtypeStruct((M * C, D), jnp.float32),
    )(char_emb)


def _sc_lookup(ids, char_table, rot, tok_emb):
    n = ids.shape[0]
    info = plsc.get_sparse_core_info()
    nw = info.num_cores * info.num_subcores
    per_w = n // nw
    T = 16
    nchunk = per_w // T
    NS = T * M // 128

    mesh = plsc.VectorSubcoreMesh(core_axis_name="c", subcore_axis_name="s")

    @functools.partial(
        pl.kernel,
        out_type=jax.ShapeDtypeStruct((n, D), jnp.float32),
        mesh=mesh,
        scratch_types=[
            pltpu.VMEM((per_w,), jnp.int32),
            [pltpu.VMEM((T * M,), jnp.int32)] * 2,
            [pltpu.VMEM((T * M,), jnp.int32)] * 2,
            [pltpu.VMEM((T * M,), jnp.int32)] * 2,
            [pltpu.VMEM((T, D), jnp.float32)] * 2,
            [pltpu.VMEM((T * M, D), jnp.float32)] * 2,
            [pltpu.SemaphoreType.DMA] * 2,
            [pltpu.SemaphoreType.DMA] * 2,
            [pltpu.SemaphoreType.DMA] * 2,
        ],
    )
    def k(ids_hbm, chart_hbm, rot_hbm, tok_hbm, out_hbm,
          ids_v, cidx, chars, rc, acc, rb, sem_c, sem_t, sem_r):
        wid = lax.axis_index("s") * info.num_cores + lax.axis_index("c")
        base = wid * per_w
        pltpu.sync_copy(ids_hbm.at[pl.ds(base, per_w)], ids_v)
        lane = lax.iota(jnp.int32, 16)
        offs = lane * C

        def fire_char(c, p):
            v = ids_v[pl.ds(c * T, 16)] * M
            for t in range(16):
                cidx[p][pl.ds(t * M, M)] = v[t] + lane
            for i in range(NS):
                sl = pl.ds(i * 128, 128)
                pltpu.async_copy(chart_hbm.at[cidx[p].at[sl]], chars[p].at[sl],
                                 sem_c[p])

        def fire_tok(c, p):
            pltpu.async_copy(tok_hbm.at[ids_v.at[pl.ds(c * T, T)]], acc[p],
                             sem_t[p])

        def fire_rot(c, p):
            for i in range(NS):
                sl = pl.ds(i * 128, 128)
                pltpu.make_async_copy(chart_hbm.at[cidx[p].at[sl]],
                                      chars[p].at[sl], sem_c[p]).wait()
            for v in range(T * M // 16):
                sl = pl.ds(v * 16, 16)
                rc[p][sl] = chars[p][sl] + offs
            for i in range(NS):
                sl = pl.ds(i * 128, 128)
                pltpu.async_copy(rot_hbm.at[rc[p].at[sl]], rb[p].at[sl],
                                 sem_r[p])

        def drain_accum(c, p):
            for i in range(NS):
                sl = pl.ds(i * 128, 128)
                pltpu.make_async_copy(rot_hbm.at[rc[p].at[sl]], rb[p].at[sl],
                                      sem_r[p]).wait()
            pltpu.make_async_copy(tok_hbm.at[ids_v.at[pl.ds(c * T, T)]],
                                  acc[p], sem_t[p]).wait()

            def tok_body(j, carry2):
                rbase = j * M
                for kk in range(D // 16):
                    sl = pl.ds(kk * 16, 16)
                    a = acc[p][j, sl]
                    for m in range(M):
                        a = a + rb[p][rbase + m, sl]
                    acc[p][j, sl] = a
                return carry2

            lax.fori_loop(0, T, tok_body, 0)
            pltpu.sync_copy(acc[p], out_hbm.at[pl.ds(base + c * T, T)])

        fire_char(0, 0)
        fire_tok(0, 0)
        fire_char(1, 1)
        fire_tok(1, 1)
        fire_rot(0, 0)

        def pair_body(c2, carry):
            c = c2 * 2
            for p in (0, 1):
                cc = c + p
                q = 1 - p
                fire_rot(cc + 1, q)
                fire_char(cc + 2, p)
                drain_accum(cc, p)
                fire_tok(cc + 2, p)
            return carry

        lax.fori_loop(0, (nchunk - 2) // 2, pair_body, 0)

        fire_rot(nchunk - 1, 1)
        drain_accum(nchunk - 2, 0)
        drain_accum(nchunk - 1, 1)

    return k(ids, char_table, rot, tok_emb)


def kernel(input, char_table, char_emb, tok_emb):
    b, s = input.shape
    rot = _rot_table(char_emb)
    ids = input.reshape(-1)
    out = _sc_lookup(ids, char_table.reshape(-1), rot, tok_emb)
    return out.reshape(b, s, D)


# speedup vs baseline: 22.2051x; 1.0144x over previous
import functools
import math

import jax
import jax.numpy as jnp
from jax import lax
from jax.experimental import pallas as pl
from jax.experimental.pallas import tpu as pltpu
from jax.experimental.pallas import tpu_sc as plsc

D = 128
M = 16
C = 256
ROPE_BASE = 10000.0


def _rot_table_body(emb_ref, out_ref):
    m = pl.program_id(0).astype(jnp.float32)
    e = emb_ref[...]
    col = lax.broadcasted_iota(jnp.int32, (C, D), 1)
    two_k = (col - (col % 2)).astype(jnp.float32)
    freq = jnp.exp(two_k * (-math.log(ROPE_BASE) / D))
    ang = m * freq
    cosv = jnp.cos(ang)
    sinv = jnp.sin(ang)
    left = jnp.concatenate([e[:, 1:], e[:, :1]], axis=1)
    right = jnp.concatenate([e[:, -1:], e[:, :-1]], axis=1)
    odd = (col % 2) == 1
    swap = jnp.where(odd, right, left)
    sign = jnp.where(odd, 1.0, -1.0)
    out_ref[...] = e * cosv + swap * sinv * sign


def _rot_table(char_emb):
    return pl.pallas_call(
        _rot_table_body,
        grid=(M,),
        in_specs=[pl.BlockSpec((C, D), lambda m: (0, 0))],
        out_specs=pl.BlockSpec((C, D), lambda m: (m, 0)),
        out_shape=jax.ShapeDtypeStruct((M * C, D), jnp.float32),
    )(char_emb)


def _sc_lookup(ids, char_table, rot, tok_emb):
    n = ids.shape[0]
    info = plsc.get_sparse_core_info()
    nw = info.num_cores * info.num_subcores
    per_w = n // nw
    T = 16
    nchunk = per_w // T
    NS = T * M // 128

    mesh = plsc.VectorSubcoreMesh(core_axis_name="c", subcore_axis_name="s")

    @functools.partial(
        pl.kernel,
        out_type=jax.ShapeDtypeStruct((n, D), jnp.float32),
        mesh=mesh,
        compiler_params=pltpu.CompilerParams(use_tc_tiling_on_sc=False),
        scratch_types=[
            pltpu.VMEM((per_w,), jnp.int32),
            [pltpu.VMEM((T, M), jnp.int32)] * 2,
            [pltpu.VMEM((T * M,), jnp.int32)] * 2,
            [pltpu.VMEM((T, D), jnp.float32)] * 2,
            [pltpu.VMEM((T * M, D), jnp.float32)] * 2,
            [pltpu.SemaphoreType.DMA] * 2,
            [pltpu.SemaphoreType.DMA] * 2,
            [pltpu.SemaphoreType.DMA] * 2,
        ],
    )
    def k(ids_hbm, chart_hbm, rot_hbm, tok_hbm, out_hbm,
          ids_v, chars, rc, acc, rb, sem_c, sem_t, sem_r):
        wid = lax.axis_index("s") * info.num_cores + lax.axis_index("c")
        base = wid * per_w
        pltpu.sync_copy(ids_hbm.at[pl.ds(base, per_w)], ids_v)
        lane = lax.iota(jnp.int32, 16)
        offs = lane * C

        def fire_char(c, p):
            pltpu.async_copy(chart_hbm.at[ids_v.at[pl.ds(c * T, T)]],
                             chars[p], sem_c[p])

        def fire_tok(c, p):
            pltpu.async_copy(tok_hbm.at[ids_v.at[pl.ds(c * T, T)]], acc[p],
                             sem_t[p])

        def fire_rot(c, p):
            pltpu.make_async_copy(chart_hbm.at[ids_v.at[pl.ds(c * T, T)]],
                                  chars[p], sem_c[p]).wait()
            for j in range(T):
                rc[p][pl.ds(j * M, M)] = chars[p][j, :] + offs
            for i in range(NS):
                sl = pl.ds(i * 128, 128)
                pltpu.async_copy(rot_hbm.at[rc[p].at[sl]], rb[p].at[sl],
                                 sem_r[p])

        def drain_accum(c, p):
            for i in range(NS):
                sl = pl.ds(i * 128, 128)
                pltpu.make_async_copy(rot_hbm.at[rc[p].at[sl]], rb[p].at[sl],
                                      sem_r[p]).wait()
            pltpu.make_async_copy(tok_hbm.at[ids_v.at[pl.ds(c * T, T)]],
                                  acc[p], sem_t[p]).wait()

            def tok_body(j, carry2):
                rbase = j * M
                for kk in range(D // 16):
                    sl = pl.ds(kk * 16, 16)
                    a = acc[p][j, sl]
                    for m in range(M):
                        a = a + rb[p][rbase + m, sl]
                    acc[p][j, sl] = a
                return carry2

            lax.fori_loop(0, T, tok_body, 0)
            pltpu.sync_copy(acc[p], out_hbm.at[pl.ds(base + c * T, T)])

        fire_char(0, 0)
        fire_tok(0, 0)
        fire_char(1, 1)
        fire_tok(1, 1)
        fire_rot(0, 0)

        def pair_body(c2, carry):
            c = c2 * 2
            for p in (0, 1):
                cc = c + p
                q = 1 - p
                fire_rot(cc + 1, q)
                fire_char(cc + 2, p)
                drain_accum(cc, p)
                fire_tok(cc + 2, p)
            return carry

        lax.fori_loop(0, (nchunk - 2) // 2, pair_body, 0)

        fire_rot(nchunk - 1, 1)
        drain_accum(nchunk - 2, 0)
        drain_accum(nchunk - 1, 1)

    return k(ids, char_table, rot, tok_emb)


def kernel(input, char_table, char_emb, tok_emb):
    b, s = input.shape
    rot = _rot_table(char_emb)
    ids = input.reshape(-1)
    out = _sc_lookup(ids, char_table, rot, tok_emb)
    return out.reshape(b, s, D)


# speedup vs baseline: 31.0721x; 1.3993x over previous
import functools
import math

import jax
import jax.numpy as jnp
from jax import lax
from jax.experimental import pallas as pl
from jax.experimental.pallas import tpu as pltpu
from jax.experimental.pallas import tpu_sc as plsc

D = 128
M = 16
C = 256
ROPE_BASE = 10000.0


def _rot_table_body(emb_ref, out_ref):
    m = pl.program_id(0).astype(jnp.float32)
    e = emb_ref[...]
    col = lax.broadcasted_iota(jnp.int32, (1, D), 1)
    u = col % 32
    ncol = (col - u) + (u % 2) * 16 + u // 2
    two_k = (ncol - (ncol % 2)).astype(jnp.float32)
    freq = jnp.exp(two_k * (-math.log(ROPE_BASE) / D))
    ang = m * freq
    cosr = jnp.cos(ang)
    sinr = jnp.sin(ang) * jnp.where((ncol % 2) == 1, 1.0, -1.0)
    rows = lax.broadcasted_iota(jnp.int32, (D, D), 0)
    p1 = (rows == ncol).astype(jnp.float32)
    p2 = (rows == (ncol ^ 1)).astype(jnp.float32)
    ep = jnp.dot(e, p1, preferred_element_type=jnp.float32)
    es = jnp.dot(e, p2, preferred_element_type=jnp.float32)
    out_ref[...] = (ep * cosr + es * sinr).astype(jnp.bfloat16)


def _rot_table(char_emb):
    return pl.pallas_call(
        _rot_table_body,
        grid=(M,),
        in_specs=[pl.BlockSpec((C, D), lambda m: (0, 0))],
        out_specs=pl.BlockSpec((C, D), lambda m: (m, 0)),
        out_shape=jax.ShapeDtypeStruct((M * C, D), jnp.bfloat16),
    )(char_emb)


def _sc_lookup(ids, char_table, rot, tok_emb):
    n = ids.shape[0]
    info = plsc.get_sparse_core_info()
    nw = info.num_cores * info.num_subcores
    per_w = n // nw
    T = 16
    nchunk = per_w // T
    NS = T * M // 128

    mesh = plsc.VectorSubcoreMesh(core_axis_name="c", subcore_axis_name="s")

    @functools.partial(
        pl.kernel,
        out_type=jax.ShapeDtypeStruct((n, D), jnp.float32),
        mesh=mesh,
        compiler_params=pltpu.CompilerParams(use_tc_tiling_on_sc=False,
                                             needs_layout_passes=False),
        scratch_types=[
            pltpu.VMEM((per_w,), jnp.int32),
            [pltpu.VMEM((T, M), jnp.int32)] * 2,
            [pltpu.VMEM((T * M,), jnp.int32)] * 2,
            [pltpu.VMEM((T, D), jnp.float32)] * 2,
            [pltpu.VMEM((T * M, D), jnp.bfloat16)] * 2,
            [pltpu.SemaphoreType.DMA] * 2,
            [pltpu.SemaphoreType.DMA] * 2,
            [pltpu.SemaphoreType.DMA] * 2,
        ],
    )
    def k(ids_hbm, chart_hbm, rot_hbm, tok_hbm, out_hbm,
          ids_v, chars, rc, acc, rb, sem_c, sem_t, sem_r):
        wid = lax.axis_index("s") * info.num_cores + lax.axis_index("c")
        base = wid * per_w
        pltpu.sync_copy(ids_hbm.at[pl.ds(base, per_w)], ids_v)
        lane = lax.iota(jnp.int32, 16)
        offs = lane * C

        def fire_char(c, p):
            pltpu.async_copy(chart_hbm.at[ids_v.at[pl.ds(c * T, T)]],
                             chars[p], sem_c[p])

        def fire_tok(c, p):
            pltpu.async_copy(tok_hbm.at[ids_v.at[pl.ds(c * T, T)]], acc[p],
                             sem_t[p])

        def fire_rot(c, p):
            pltpu.make_async_copy(chart_hbm.at[ids_v.at[pl.ds(c * T, T)]],
                                  chars[p], sem_c[p]).wait()
            for j in range(T):
                rc[p][pl.ds(j * M, M)] = chars[p][j, :] + offs
            for i in range(NS):
                sl = pl.ds(i * 128, 128)
                pltpu.async_copy(rot_hbm.at[rc[p].at[sl]], rb[p].at[sl],
                                 sem_r[p])

        def drain_accum(c, p):
            for i in range(NS):
                sl = pl.ds(i * 128, 128)
                pltpu.make_async_copy(rot_hbm.at[rc[p].at[sl]], rb[p].at[sl],
                                      sem_r[p]).wait()
            pltpu.make_async_copy(tok_hbm.at[ids_v.at[pl.ds(c * T, T)]],
                                  acc[p], sem_t[p]).wait()

            def tok_body(j, carry2):
                rbase = j * M
                for g in range(D // 32):
                    sla = pl.ds(g * 32, 16)
                    slb = pl.ds(g * 32 + 16, 16)
                    a = acc[p][j, sla]
                    b = acc[p][j, slb]
                    for m in range(M):
                        raw = rb[p][rbase + m, pl.ds(g * 32, 32)]
                        x, y = plsc.unpack(raw,
                                           format=plsc.PackFormat.INTERLEAVED)
                        a = a + x
                        b = b + y
                    acc[p][j, sla] = a
                    acc[p][j, slb] = b
                return carry2

            lax.fori_loop(0, T, tok_body, 0)
            pltpu.sync_copy(acc[p], out_hbm.at[pl.ds(base + c * T, T)])

        fire_char(0, 0)
        fire_tok(0, 0)
        fire_char(1, 1)
        fire_tok(1, 1)
        fire_rot(0, 0)

        def pair_body(c2, carry):
            c = c2 * 2
            for p in (0, 1):
                cc = c + p
                q = 1 - p
                fire_rot(cc + 1, q)
                fire_char(cc + 2, p)
                drain_accum(cc, p)
                fire_tok(cc + 2, p)
            return carry

        lax.fori_loop(0, (nchunk - 2) // 2, pair_body, 0)

        fire_rot(nchunk - 1, 1)
        drain_accum(nchunk - 2, 0)
        drain_accum(nchunk - 1, 1)

    return k(ids, char_table, rot, tok_emb)


def kernel(input, char_table, char_emb, tok_emb):
    b, s = input.shape
    rot = _rot_table(char_emb)
    ids = input.reshape(-1)
    out = _sc_lookup(ids, char_table, rot, tok_emb)
    return out.reshape(b, s, D)


# speedup vs baseline: 31.1187x; 1.0015x over previous
import functools
import math

import jax
import jax.numpy as jnp
from jax import lax
from jax.experimental import pallas as pl
from jax.experimental.pallas import tpu as pltpu
from jax.experimental.pallas import tpu_sc as plsc

D = 128
M = 16
C = 256
ROPE_BASE = 10000.0


def _rot_table_body(emb_ref, out_ref):
    m = pl.program_id(0).astype(jnp.float32)
    e = emb_ref[...]
    col = lax.broadcasted_iota(jnp.int32, (1, D), 1)
    u = col % 32
    ncol = (col - u) + (u % 2) * 16 + u // 2
    two_k = (ncol - (ncol % 2)).astype(jnp.float32)
    freq = jnp.exp(two_k * (-math.log(ROPE_BASE) / D))
    ang = m * freq
    cosr = jnp.cos(ang)
    sinr = jnp.sin(ang) * jnp.where((ncol % 2) == 1, 1.0, -1.0)
    rows = lax.broadcasted_iota(jnp.int32, (D, D), 0)
    p1 = (rows == ncol).astype(jnp.float32)
    p2 = (rows == (ncol ^ 1)).astype(jnp.float32)
    ep = jnp.dot(e, p1, preferred_element_type=jnp.float32)
    es = jnp.dot(e, p2, preferred_element_type=jnp.float32)
    out_ref[...] = (ep * cosr + es * sinr).astype(jnp.bfloat16)


def _rot_table(char_emb):
    return pl.pallas_call(
        _rot_table_body,
        grid=(M,),
        in_specs=[pl.BlockSpec((C, D), lambda m: (0, 0))],
        out_specs=pl.BlockSpec((C, D), lambda m: (m, 0)),
        out_shape=jax.ShapeDtypeStruct((M * C, D), jnp.bfloat16),
    )(char_emb)


def _sc_lookup(ids, char_table, rot, tok_emb):
    b, s = ids.shape
    n = b * s
    info = plsc.get_sparse_core_info()
    nw = info.num_cores * info.num_subcores
    per_w = n // nw
    wpr = s // per_w
    T = 16
    nchunk = per_w // T
    NS = T * M // 128

    mesh = plsc.VectorSubcoreMesh(core_axis_name="c", subcore_axis_name="s")

    @functools.partial(
        pl.kernel,
        out_type=jax.ShapeDtypeStruct((b, s, D), jnp.float32),
        mesh=mesh,
        compiler_params=pltpu.CompilerParams(use_tc_tiling_on_sc=False,
                                             needs_layout_passes=False,
                                             disable_bounds_checks=True),
        scratch_types=[
            pltpu.VMEM((per_w,), jnp.int32),
            [pltpu.VMEM((T, M), jnp.int32)] * 2,
            [pltpu.VMEM((T * M,), jnp.int32)] * 2,
            [pltpu.VMEM((T, D), jnp.float32)] * 2,
            [pltpu.VMEM((T * M, D), jnp.bfloat16)] * 2,
            [pltpu.SemaphoreType.DMA] * 2,
            [pltpu.SemaphoreType.DMA] * 2,
            [pltpu.SemaphoreType.DMA] * 2,
        ],
    )
    def k(ids_hbm, chart_hbm, rot_hbm, tok_hbm, out_hbm,
          ids_v, chars, rc, acc, rb, sem_c, sem_t, sem_r):
        wid = lax.axis_index("s") * info.num_cores + lax.axis_index("c")
        row = wid // wpr
        s0 = (wid % wpr) * per_w
        pltpu.sync_copy(ids_hbm.at[row, pl.ds(s0, per_w)], ids_v)
        lane = lax.iota(jnp.int32, 16)
        offs = lane * C

        def fire_char(c, p):
            pltpu.async_copy(chart_hbm.at[ids_v.at[pl.ds(c * T, T)]],
                             chars[p], sem_c[p])

        def fire_tok(c, p):
            pltpu.async_copy(tok_hbm.at[ids_v.at[pl.ds(c * T, T)]], acc[p],
                             sem_t[p])

        def fire_rot(c, p):
            pltpu.make_async_copy(chart_hbm.at[ids_v.at[pl.ds(c * T, T)]],
                                  chars[p], sem_c[p]).wait()
            for j in range(T):
                rc[p][pl.ds(j * M, M)] = chars[p][j, :] + offs
            for i in range(NS):
                sl = pl.ds(i * 128, 128)
                pltpu.async_copy(rot_hbm.at[rc[p].at[sl]], rb[p].at[sl],
                                 sem_r[p])

        def drain_accum(c, p):
            for i in range(NS):
                sl = pl.ds(i * 128, 128)
                pltpu.make_async_copy(rot_hbm.at[rc[p].at[sl]], rb[p].at[sl],
                                      sem_r[p]).wait()
            pltpu.make_async_copy(tok_hbm.at[ids_v.at[pl.ds(c * T, T)]],
                                  acc[p], sem_t[p]).wait()

            def tok_body(j, carry2):
                rbase = j * M
                for g in range(D // 32):
                    sla = pl.ds(g * 32, 16)
                    slb = pl.ds(g * 32 + 16, 16)
                    a = acc[p][j, sla]
                    b = acc[p][j, slb]
                    for m in range(M):
                        raw = rb[p][rbase + m, pl.ds(g * 32, 32)]
                        x, y = plsc.unpack(raw,
                                           format=plsc.PackFormat.INTERLEAVED)
                        a = a + x
                        b = b + y
                    acc[p][j, sla] = a
                    acc[p][j, slb] = b
                return carry2

            lax.fori_loop(0, T, tok_body, 0)
            pltpu.sync_copy(acc[p], out_hbm.at[row, pl.ds(s0 + c * T, T), :])

        fire_char(0, 0)
        fire_tok(0, 0)
        fire_char(1, 1)
        fire_tok(1, 1)
        fire_rot(0, 0)

        def pair_body(c2, carry):
            c = c2 * 2
            for p in (0, 1):
                cc = c + p
                q = 1 - p
                fire_rot(cc + 1, q)
                fire_char(cc + 2, p)
                drain_accum(cc, p)
                fire_tok(cc + 2, p)
            return carry

        lax.fori_loop(0, (nchunk - 2) // 2, pair_body, 0)

        fire_rot(nchunk - 1, 1)
        drain_accum(nchunk - 2, 0)
        drain_accum(nchunk - 1, 1)

    return k(ids, char_table, rot, tok_emb)


def kernel(input, char_table, char_emb, tok_emb):
    rot = _rot_table(char_emb)
    return _sc_lookup(input, char_table, rot, tok_emb)
